# Initial kernel scaffold; baseline (speedup 1.0000x reference)
import jax
import jax.numpy as jnp
from jax.experimental import pallas as pl


def kernel(x, table):
    raise NotImplementedError("write your pallas kernel here")


# speedup vs baseline: 6.5359x; 6.5359x over previous
import functools

import jax
import jax.numpy as jnp
from jax import lax
from jax.experimental import pallas as pl
from jax.experimental.pallas import tpu as pltpu
from jax.experimental.pallas import tpu_sc as plsc

VOCAB = 1000
D_MODEL = 128
BATCH = 4096
HIST = 200

_NC = 2
_NS = 16
_NW = _NC * _NS

_B = BATCH * HIST
_B_PER_W = _B // _NW
_CHUNK = 128
_N_CHUNKS = _B_PER_W // _CHUNK


def _gather_kernel(idx_hbm, table_hbm, out_hbm, idx_v, rows_a, rows_b, sem):
    wid = lax.axis_index("s") * _NC + lax.axis_index("c")
    base = wid * _B_PER_W

    pltpu.sync_copy(idx_hbm.at[wid], idx_v)

    def gather(i, buf):
        return pltpu.async_copy(table_hbm.at[idx_v.at[i]], buf, sem)

    gather(0, rows_a)

    def body(g, _):
        i0 = 2 * g
        gather(i0 + 1, rows_b)
        pltpu.make_async_copy(table_hbm.at[idx_v.at[i0]], rows_a, sem).wait()
        pltpu.sync_copy(rows_a, out_hbm.at[pl.ds(base + i0 * _CHUNK, _CHUNK)])

        @pl.when(i0 + 2 < _N_CHUNKS)
        def _():
            gather(i0 + 2, rows_a)

        pltpu.make_async_copy(table_hbm.at[idx_v.at[i0 + 1]], rows_b, sem).wait()
        pltpu.sync_copy(rows_b, out_hbm.at[pl.ds(base + (i0 + 1) * _CHUNK, _CHUNK)])
        return _

    lax.fori_loop(0, _N_CHUNKS // 2, body, None)


@functools.partial(jax.jit, donate_argnums=())
def _embed(x, table):
    idx = x.reshape(_NW, _N_CHUNKS, _CHUNK)
    mesh = plsc.VectorSubcoreMesh(core_axis_name="c", subcore_axis_name="s")
    out = pl.kernel(
        _gather_kernel,
        mesh=mesh,
        out_type=jax.ShapeDtypeStruct((_B, D_MODEL), jnp.float32),
        scratch_types=[
            pltpu.VMEM((_N_CHUNKS, _CHUNK), jnp.int32),
            pltpu.VMEM((_CHUNK, D_MODEL), jnp.float32),
            pltpu.VMEM((_CHUNK, D_MODEL), jnp.float32),
            pltpu.SemaphoreType.DMA,
        ],
    )(idx, table)
    return out.reshape(BATCH, HIST, D_MODEL)


def kernel(x, table):
    return _embed(x, table)


# speedup vs baseline: 15.7347x; 2.4074x over previous
import functools

import jax
import jax.numpy as jnp
from jax import lax
from jax.experimental import pallas as pl
from jax.experimental.pallas import tpu as pltpu
from jax.experimental.pallas import tpu_sc as plsc

VOCAB = 1000
D_MODEL = 128
BATCH = 4096
HIST = 200

_NC = 2
_NS = 16
_NW = _NC * _NS

_B = BATCH * HIST
_B_PER_W = _B // _NW
_CHUNK = 128
_N_CHUNKS = _B_PER_W // _CHUNK


def _gather_kernel(idx_hbm, table_hbm, out_hbm, idx_v, rows_a, rows_b, table_sh, sem):
    sid = lax.axis_index("s")
    wid = sid * _NC + lax.axis_index("c")
    base = wid * _B_PER_W

    @pl.when(sid == 0)
    def _():
        pltpu.sync_copy(table_hbm, table_sh)

    pltpu.sync_copy(idx_hbm.at[wid], idx_v)
    plsc.subcore_barrier()

    def gather(i, buf):
        return pltpu.async_copy(table_sh.at[idx_v.at[i]], buf, sem)

    gather(0, rows_a)

    def body(g, _):
        i0 = 2 * g
        gather(i0 + 1, rows_b)
        pltpu.make_async_copy(table_sh.at[idx_v.at[i0]], rows_a, sem).wait()
        pltpu.sync_copy(rows_a, out_hbm.at[pl.ds(base + i0 * _CHUNK, _CHUNK)])

        @pl.when(i0 + 2 < _N_CHUNKS)
        def _():
            gather(i0 + 2, rows_a)

        pltpu.make_async_copy(table_sh.at[idx_v.at[i0 + 1]], rows_b, sem).wait()
        pltpu.sync_copy(rows_b, out_hbm.at[pl.ds(base + (i0 + 1) * _CHUNK, _CHUNK)])
        return _

    lax.fori_loop(0, _N_CHUNKS // 2, body, None)


@functools.partial(jax.jit, donate_argnums=())
def _embed(x, table):
    idx = x.reshape(_NW, _N_CHUNKS, _CHUNK)
    mesh = plsc.VectorSubcoreMesh(core_axis_name="c", subcore_axis_name="s")
    out = pl.kernel(
        _gather_kernel,
        mesh=mesh,
        out_type=jax.ShapeDtypeStruct((_B, D_MODEL), jnp.float32),
        scratch_types=[
            pltpu.VMEM((_N_CHUNKS, _CHUNK), jnp.int32),
            pltpu.VMEM((_CHUNK, D_MODEL), jnp.float32),
            pltpu.VMEM((_CHUNK, D_MODEL), jnp.float32),
            pltpu.VMEM_SHARED((VOCAB, D_MODEL), jnp.float32),
            pltpu.SemaphoreType.DMA,
        ],
    )(idx, table)
    return out.reshape(BATCH, HIST, D_MODEL)


def kernel(x, table):
    return _embed(x, table)


# speedup vs baseline: 15.9995x; 1.0168x over previous
import functools

import jax
import jax.numpy as jnp
from jax import lax
from jax.experimental import pallas as pl
from jax.experimental.pallas import tpu as pltpu
from jax.experimental.pallas import tpu_sc as plsc

VOCAB = 1000
D_MODEL = 128
BATCH = 4096
HIST = 200

_NC = 2
_NS = 16
_NW = _NC * _NS

_B = BATCH * HIST
_B_PER_W = _B // _NW
_G = 128
_CHUNK = 256
_N_CHUNKS = _B_PER_W // _CHUNK


def _gather_kernel(idx_hbm, table_hbm, out_hbm, idx_v, rows_a, rows_b, table_sh, sem):
    sid = lax.axis_index("s")
    wid = sid * _NC + lax.axis_index("c")
    base = wid * _B_PER_W

    @pl.when(sid == 0)
    def _():
        pltpu.sync_copy(table_hbm, table_sh)

    pltpu.sync_copy(idx_hbm.at[wid], idx_v)
    plsc.subcore_barrier()

    def gather(i, buf):
        for h in range(_CHUNK // _G):
            pltpu.async_copy(
                table_sh.at[idx_v.at[i, pl.ds(h * _G, _G)]],
                buf.at[pl.ds(h * _G, _G)],
                sem,
            )

    def wait_gather(i, buf):
        for h in range(_CHUNK // _G):
            pltpu.make_async_copy(
                table_sh.at[idx_v.at[i, pl.ds(h * _G, _G)]],
                buf.at[pl.ds(h * _G, _G)],
                sem,
            ).wait()

    gather(0, rows_a)

    def body(g, _):
        i0 = 2 * g
        gather(i0 + 1, rows_b)
        wait_gather(i0, rows_a)
        pltpu.sync_copy(rows_a, out_hbm.at[pl.ds(base + i0 * _CHUNK, _CHUNK)])

        @pl.when(i0 + 2 < _N_CHUNKS)
        def _():
            gather(i0 + 2, rows_a)

        wait_gather(i0 + 1, rows_b)
        pltpu.sync_copy(rows_b, out_hbm.at[pl.ds(base + (i0 + 1) * _CHUNK, _CHUNK)])
        return _

    lax.fori_loop(0, _N_CHUNKS // 2, body, None)


@functools.partial(jax.jit, donate_argnums=())
def _embed(x, table):
    idx = x.reshape(_NW, _N_CHUNKS, _CHUNK)
    assert _CHUNK % _G == 0 and _B_PER_W % _CHUNK == 0 and _N_CHUNKS % 2 == 0
    mesh = plsc.VectorSubcoreMesh(core_axis_name="c", subcore_axis_name="s")
    out = pl.kernel(
        _gather_kernel,
        mesh=mesh,
        out_type=jax.ShapeDtypeStruct((_B, D_MODEL), jnp.float32),
        scratch_types=[
            pltpu.VMEM((_N_CHUNKS, _CHUNK), jnp.int32),
            pltpu.VMEM((_CHUNK, D_MODEL), jnp.float32),
            pltpu.VMEM((_CHUNK, D_MODEL), jnp.float32),
            pltpu.VMEM_SHARED((VOCAB, D_MODEL), jnp.float32),
            pltpu.SemaphoreType.DMA,
        ],
    )(idx, table)
    return out.reshape(BATCH, HIST, D_MODEL)


def kernel(x, table):
    return _embed(x, table)


# speedup vs baseline: 16.1083x; 1.0068x over previous
import functools

import jax
import jax.numpy as jnp
from jax import lax
from jax.experimental import pallas as pl
from jax.experimental.pallas import tpu as pltpu
from jax.experimental.pallas import tpu_sc as plsc

VOCAB = 1000
D_MODEL = 128
BATCH = 4096
HIST = 200

_NC = 2
_NS = 16
_NW = _NC * _NS

_B = BATCH * HIST
_B_PER_W = _B // _NW
_G = 128
_N = _B_PER_W // _G
_NBUF = 4


def _gather_kernel(idx_hbm, table_hbm, out_hbm, idx_v, bufs, table_sh, gsem, ssem):
    sid = lax.axis_index("s")
    wid = sid * _NC + lax.axis_index("c")
    base = wid * _B_PER_W

    @pl.when(sid == 0)
    def _():
        pltpu.sync_copy(table_hbm, table_sh)

    pltpu.sync_copy(idx_hbm.at[wid], idx_v)
    plsc.subcore_barrier()

    def gather(i, j):
        pltpu.async_copy(table_sh.at[idx_v.at[i]], bufs.at[j], gsem)

    def wait_gather(i, j):
        pltpu.make_async_copy(table_sh.at[idx_v.at[i]], bufs.at[j], gsem).wait()

    def store(i, j):
        pltpu.async_copy(bufs.at[j], out_hbm.at[pl.ds(base + i * _G, _G)], ssem)

    def wait_store(i, j):
        pltpu.make_async_copy(
            bufs.at[j], out_hbm.at[pl.ds(base + i * _G, _G)], ssem
        ).wait()

    for j in range(_NBUF - 1):
        gather(j, j)

    def body(g, _):
        for j in range(_NBUF):
            i = _NBUF * g + j
            wait_gather(i, j)
            store(i, j)

            @pl.when(i > 0)
            def _():
                wait_store(i - 1, (j - 1) % _NBUF)

            @pl.when(i + _NBUF - 1 < _N)
            def _():
                gather(i + _NBUF - 1, (j - 1) % _NBUF)

        return _

    lax.fori_loop(0, _N // _NBUF, body, None)
    wait_store(_N - 1, (_N - 1) % _NBUF)


@functools.partial(jax.jit, donate_argnums=())
def _embed(x, table):
    assert _B_PER_W % _G == 0 and _N % _NBUF == 0
    idx = x.reshape(_NW, _N, _G)
    mesh = plsc.VectorSubcoreMesh(core_axis_name="c", subcore_axis_name="s")
    out = pl.kernel(
        _gather_kernel,
        mesh=mesh,
        out_type=jax.ShapeDtypeStruct((_B, D_MODEL), jnp.float32),
        scratch_types=[
            pltpu.VMEM((_N, _G), jnp.int32),
            pltpu.VMEM((_NBUF, _G, D_MODEL), jnp.float32),
            pltpu.VMEM_SHARED((VOCAB, D_MODEL), jnp.float32),
            pltpu.SemaphoreType.DMA,
            pltpu.SemaphoreType.DMA,
        ],
    )(idx, table)
    return out.reshape(BATCH, HIST, D_MODEL)


def kernel(x, table):
    return _embed(x, table)
